# baseline (speedup 1.0000x reference)
import functools

import jax
import jax.numpy as jnp
from jax import lax
from jax.experimental import pallas as pl
from jax.experimental.pallas import tpu as pltpu
from jax.experimental.pallas import tpu_sc as plsc

_N = 320000
_D = 128
_H = 256
_S = 10000

_NC = 2
_NS = 16
_NW = _NC * _NS
_CH = 80
_BN = 2560

_SLABS = (60 * _BN, 40 * _BN, 25 * _BN)

_NBUF = 4
_LEAD = 2
_SP = 10112
_ZR = _SP // _NS


def _pn_body(x_ref, w1_ref, b1_ref, w2_ref, b2_ref, o_ref):
    h = jnp.maximum(
        jnp.dot(x_ref[...], w1_ref[...], preferred_element_type=jnp.float32)
        + b1_ref[...],
        0.0,
    )
    o_ref[...] = (
        jnp.dot(h, w2_ref[...], preferred_element_type=jnp.float32) + b2_ref[...]
    )


def _point_net(x, w1, b1, w2, b2, start, n):
    off = start // _BN
    return pl.pallas_call(
        _pn_body,
        grid=(n // _BN,),
        in_specs=[
            pl.BlockSpec((_BN, _D), lambda i: (i + off, 0)),
            pl.BlockSpec((_D, _H), lambda i: (0, 0)),
            pl.BlockSpec((1, _H), lambda i: (0, 0)),
            pl.BlockSpec((_H, _D), lambda i: (0, 0)),
            pl.BlockSpec((1, _D), lambda i: (0, 0)),
        ],
        out_specs=pl.BlockSpec((_BN, _D), lambda i: (i, 0)),
        out_shape=jax.ShapeDtypeStruct((n, _D), jnp.float32),
    )(x, w1, b1, w2, b2)


_BS = 2000


def _rn_body(*refs):
    p_refs, (w1_ref, b1_ref, w2_ref, b2_ref, o_ref) = refs[:-5], refs[-5:]
    seg = sum(p[0] + p[1] for p in p_refs)
    h = jnp.maximum(
        jnp.dot(seg, w1_ref[...], preferred_element_type=jnp.float32) + b1_ref[...],
        0.0,
    )
    o_ref[...] = (
        jnp.dot(h, w2_ref[...], preferred_element_type=jnp.float32) + b2_ref[...]
    )


def _reduce_net(partials, w1, b1, w2, b2):
    return pl.pallas_call(
        _rn_body,
        grid=(_S // _BS,),
        in_specs=(
            [pl.BlockSpec((2, _BS, _D), lambda i: (0, i, 0)) for _ in partials]
            + [
                pl.BlockSpec((_D, _H), lambda i: (0, 0)),
                pl.BlockSpec((1, _H), lambda i: (0, 0)),
                pl.BlockSpec((_H, _D), lambda i: (0, 0)),
                pl.BlockSpec((1, _D), lambda i: (0, 0)),
            ]
        ),
        out_specs=pl.BlockSpec((_BS, _D), lambda i: (i, 0)),
        out_shape=jax.ShapeDtypeStruct((_S, _D), jnp.float32),
    )(*partials, w1, b1, w2, b2)


@functools.cache
def _make_seg_sum(nch):
    rpw = nch * _CH
    mesh = plsc.VectorSubcoreMesh(core_axis_name="c", subcore_axis_name="s")

    @functools.partial(
        pl.kernel,
        mesh=mesh,
        out_type=jax.ShapeDtypeStruct((_NC, _SP, _D), jnp.float32),
        scratch_types=(
            [pltpu.VMEM((1, _CH), jnp.int32) for _ in range(_NBUF)]
            + [pltpu.VMEM((_CH, _D), jnp.float32) for _ in range(_NBUF)]
            + [pltpu.VMEM_SHARED((_SP, _D), jnp.float32)]
            + [pltpu.SemaphoreType.DMA for _ in range(2 * _NBUF)]
        ),
    )
    def seg_sum(pt_hbm, idx3_hbm, zrows_hbm, out_hbm, *scr):
        ibufs = scr[:_NBUF]
        rbufs = scr[_NBUF : 2 * _NBUF]
        seg_sh = scr[2 * _NBUF]
        lsems = scr[2 * _NBUF + 1 : 3 * _NBUF + 1]
        ssems = scr[3 * _NBUF + 1 : 4 * _NBUF + 1]
        c = lax.axis_index("c")
        s = lax.axis_index("s")
        wid = c * _NS + s
        pltpu.sync_copy(zrows_hbm, seg_sh.at[pl.ds(s * _ZR, _ZR)])
        plsc.subcore_barrier()

        base = wid * rpw

        def rows_src(j):
            return pt_hbm.at[pl.ds(base + j * _CH, _CH)]

        def start_loads(j):
            b = j % _NBUF
            pltpu.async_copy(idx3_hbm.at[wid, j], ibufs[b], lsems[b])
            pltpu.async_copy(rows_src(j), rbufs[b], lsems[b])

        def wait_loads(j):
            b = j % _NBUF
            pltpu.make_async_copy(idx3_hbm.at[wid, j], ibufs[b], lsems[b]).wait()
            pltpu.make_async_copy(rows_src(j), rbufs[b], lsems[b]).wait()

        def drain_scatter(j):
            b = j % _NBUF
            pltpu.make_async_copy(rows_src(j), rbufs[b], ssems[b]).wait()

        for j in range(_LEAD):
            start_loads(j)
        for j in range(nch):
            b = j % _NBUF
            wait_loads(j)
            pltpu.async_copy(
                rbufs[b], seg_sh.at[ibufs[b].at[0]], ssems[b], add=True
            )
            if j >= _LEAD:
                drain_scatter(j - _LEAD)
            if j + _LEAD < nch:
                start_loads(j + _LEAD)
        for j in range(nch - _LEAD, nch):
            drain_scatter(j)
        plsc.subcore_barrier()
        pltpu.sync_copy(
            seg_sh.at[pl.ds(s * _ZR, _ZR)], out_hbm.at[c, pl.ds(s * _ZR, _ZR)]
        )

    return seg_sum


def kernel(x, idx, W1p, b1p, W2p, b2p, W1r, b1r, W2r, b2r):
    idx32 = idx.astype(jnp.int32)
    b1p2, b2p2 = b1p.reshape(1, _H), b2p.reshape(1, _D)
    zrows = jnp.zeros((_ZR, _D), jnp.float32)
    partials = []
    start = 0
    for n in _SLABS:
        nch = n // (_NW * _CH)
        idx3 = lax.slice_in_dim(idx32, start, start + n).reshape(
            _NW, nch, 1, _CH
        )
        pt = _point_net(x, W1p, b1p2, W2p, b2p2, start, n)
        partials.append(_make_seg_sum(nch)(pt, idx3, zrows))
        start += n
    return _reduce_net(
        partials, W1r, b1r.reshape(1, _H), W2r, b2r.reshape(1, _D)
    )


# speedup vs baseline: 1.0584x; 1.0584x over previous
import functools

import jax
import jax.numpy as jnp
from jax import lax
from jax.experimental import pallas as pl
from jax.experimental.pallas import tpu as pltpu
from jax.experimental.pallas import tpu_sc as plsc

_N = 320000
_D = 128
_H = 256
_S = 10000

_NC = 2
_NS = 16
_NW = _NC * _NS
_CH = 80
_BN = 6400

_SLABS = (_N,)

_NBUF = 4
_LEAD = 2
_SP = 10112
_ZR = _SP // _NS


def _pn_body(x_ref, w1_ref, b1_ref, w2_ref, b2_ref, o_ref):
    h = jnp.maximum(
        jnp.dot(x_ref[...], w1_ref[...], preferred_element_type=jnp.float32)
        + b1_ref[...],
        0.0,
    )
    o_ref[...] = (
        jnp.dot(h, w2_ref[...], preferred_element_type=jnp.float32) + b2_ref[...]
    )


def _point_net(x, w1, b1, w2, b2, start, n):
    off = start // _BN
    return pl.pallas_call(
        _pn_body,
        grid=(n // _BN,),
        in_specs=[
            pl.BlockSpec((_BN, _D), lambda i: (i + off, 0)),
            pl.BlockSpec((_D, _H), lambda i: (0, 0)),
            pl.BlockSpec((1, _H), lambda i: (0, 0)),
            pl.BlockSpec((_H, _D), lambda i: (0, 0)),
            pl.BlockSpec((1, _D), lambda i: (0, 0)),
        ],
        out_specs=pl.BlockSpec((_BN, _D), lambda i: (i, 0)),
        out_shape=jax.ShapeDtypeStruct((n, _D), jnp.float32),
    )(x, w1, b1, w2, b2)


_BS = 2000


def _rn_body(*refs):
    p_refs, (w1_ref, b1_ref, w2_ref, b2_ref, o_ref) = refs[:-5], refs[-5:]
    seg = sum(p[0] + p[1] for p in p_refs)
    h = jnp.maximum(
        jnp.dot(seg, w1_ref[...], preferred_element_type=jnp.float32) + b1_ref[...],
        0.0,
    )
    o_ref[...] = (
        jnp.dot(h, w2_ref[...], preferred_element_type=jnp.float32) + b2_ref[...]
    )


def _reduce_net(partials, w1, b1, w2, b2):
    return pl.pallas_call(
        _rn_body,
        grid=(_S // _BS,),
        in_specs=(
            [pl.BlockSpec((2, _BS, _D), lambda i: (0, i, 0)) for _ in partials]
            + [
                pl.BlockSpec((_D, _H), lambda i: (0, 0)),
                pl.BlockSpec((1, _H), lambda i: (0, 0)),
                pl.BlockSpec((_H, _D), lambda i: (0, 0)),
                pl.BlockSpec((1, _D), lambda i: (0, 0)),
            ]
        ),
        out_specs=pl.BlockSpec((_BS, _D), lambda i: (i, 0)),
        out_shape=jax.ShapeDtypeStruct((_S, _D), jnp.float32),
    )(*partials, w1, b1, w2, b2)


@functools.cache
def _make_seg_sum(nch):
    rpw = nch * _CH
    mesh = plsc.VectorSubcoreMesh(core_axis_name="c", subcore_axis_name="s")

    @functools.partial(
        pl.kernel,
        mesh=mesh,
        out_type=jax.ShapeDtypeStruct((_NC, _SP, _D), jnp.float32),
        scratch_types=(
            [pltpu.VMEM((1, _CH), jnp.int32) for _ in range(_NBUF)]
            + [pltpu.VMEM((_CH, _D), jnp.float32) for _ in range(_NBUF)]
            + [pltpu.VMEM_SHARED((_SP, _D), jnp.float32)]
            + [pltpu.SemaphoreType.DMA for _ in range(2 * _NBUF)]
        ),
    )
    def seg_sum(pt_hbm, idx3_hbm, zrows_hbm, out_hbm, *scr):
        ibufs = scr[:_NBUF]
        rbufs = scr[_NBUF : 2 * _NBUF]
        seg_sh = scr[2 * _NBUF]
        lsems = scr[2 * _NBUF + 1 : 3 * _NBUF + 1]
        ssems = scr[3 * _NBUF + 1 : 4 * _NBUF + 1]
        c = lax.axis_index("c")
        s = lax.axis_index("s")
        wid = c * _NS + s
        pltpu.sync_copy(zrows_hbm, seg_sh.at[pl.ds(s * _ZR, _ZR)])
        plsc.subcore_barrier()

        base = wid * rpw

        def rows_src(j):
            return pt_hbm.at[pl.ds(base + j * _CH, _CH)]

        def start_loads(j):
            b = j % _NBUF
            pltpu.async_copy(idx3_hbm.at[wid, j], ibufs[b], lsems[b])
            pltpu.async_copy(rows_src(j), rbufs[b], lsems[b])

        def wait_loads(j):
            b = j % _NBUF
            pltpu.make_async_copy(idx3_hbm.at[wid, j], ibufs[b], lsems[b]).wait()
            pltpu.make_async_copy(rows_src(j), rbufs[b], lsems[b]).wait()

        def drain_scatter(j):
            b = j % _NBUF
            pltpu.make_async_copy(rows_src(j), rbufs[b], ssems[b]).wait()

        for j in range(_LEAD):
            start_loads(j)
        for j in range(nch):
            b = j % _NBUF
            wait_loads(j)
            pltpu.async_copy(
                rbufs[b], seg_sh.at[ibufs[b].at[0]], ssems[b], add=True
            )
            if j >= _LEAD:
                drain_scatter(j - _LEAD)
            if j + _LEAD < nch:
                start_loads(j + _LEAD)
        for j in range(nch - _LEAD, nch):
            drain_scatter(j)
        plsc.subcore_barrier()
        pltpu.sync_copy(
            seg_sh.at[pl.ds(s * _ZR, _ZR)], out_hbm.at[c, pl.ds(s * _ZR, _ZR)]
        )

    return seg_sum


def kernel(x, idx, W1p, b1p, W2p, b2p, W1r, b1r, W2r, b2r):
    idx32 = idx.astype(jnp.int32)
    b1p2, b2p2 = b1p.reshape(1, _H), b2p.reshape(1, _D)
    zrows = jnp.zeros((_ZR, _D), jnp.float32)
    partials = []
    start = 0
    for n in _SLABS:
        nch = n // (_NW * _CH)
        idx3 = lax.slice_in_dim(idx32, start, start + n).reshape(
            _NW, nch, 1, _CH
        )
        pt = _point_net(x, W1p, b1p2, W2p, b2p2, start, n)
        partials.append(_make_seg_sum(nch)(pt, idx3, zrows))
        start += n
    return _reduce_net(
        partials, W1r, b1r.reshape(1, _H), W2r, b2r.reshape(1, _D)
    )


# speedup vs baseline: 1.0864x; 1.0264x over previous
import functools

import jax
import jax.numpy as jnp
from jax import lax
from jax.experimental import pallas as pl
from jax.experimental.pallas import tpu as pltpu
from jax.experimental.pallas import tpu_sc as plsc

_N = 320000
_D = 128
_H = 256
_S = 10000

_NC = 2
_NS = 16
_NW = _NC * _NS
_CH = 80
_BN = 8000

_SLABS = (_N,)

_NBUF = 4
_LEAD = 2
_SP = 10112
_ZR = _SP // _NS


def _pn_body(x_ref, w1_ref, b1_ref, w2_ref, b2_ref, o_ref):
    h = jnp.maximum(
        jnp.dot(x_ref[...], w1_ref[...], preferred_element_type=jnp.float32)
        + b1_ref[...],
        0.0,
    )
    o_ref[...] = (
        jnp.dot(h, w2_ref[...], preferred_element_type=jnp.float32) + b2_ref[...]
    )


def _point_net(x, w1, b1, w2, b2, start, n):
    off = start // _BN
    return pl.pallas_call(
        _pn_body,
        grid=(n // _BN,),
        in_specs=[
            pl.BlockSpec((_BN, _D), lambda i: (i + off, 0)),
            pl.BlockSpec((_D, _H), lambda i: (0, 0)),
            pl.BlockSpec((1, _H), lambda i: (0, 0)),
            pl.BlockSpec((_H, _D), lambda i: (0, 0)),
            pl.BlockSpec((1, _D), lambda i: (0, 0)),
        ],
        out_specs=pl.BlockSpec((_BN, _D), lambda i: (i, 0)),
        out_shape=jax.ShapeDtypeStruct((n, _D), jnp.float32),
    )(x, w1, b1, w2, b2)


_BS = 2000


def _rn_body(*refs):
    p_refs, (w1_ref, b1_ref, w2_ref, b2_ref, o_ref) = refs[:-5], refs[-5:]
    seg = sum(p[0] + p[1] for p in p_refs)
    h = jnp.maximum(
        jnp.dot(seg, w1_ref[...], preferred_element_type=jnp.float32) + b1_ref[...],
        0.0,
    )
    o_ref[...] = (
        jnp.dot(h, w2_ref[...], preferred_element_type=jnp.float32) + b2_ref[...]
    )


def _reduce_net(partials, w1, b1, w2, b2):
    return pl.pallas_call(
        _rn_body,
        grid=(_S // _BS,),
        in_specs=(
            [pl.BlockSpec((2, _BS, _D), lambda i: (0, i, 0)) for _ in partials]
            + [
                pl.BlockSpec((_D, _H), lambda i: (0, 0)),
                pl.BlockSpec((1, _H), lambda i: (0, 0)),
                pl.BlockSpec((_H, _D), lambda i: (0, 0)),
                pl.BlockSpec((1, _D), lambda i: (0, 0)),
            ]
        ),
        out_specs=pl.BlockSpec((_BS, _D), lambda i: (i, 0)),
        out_shape=jax.ShapeDtypeStruct((_S, _D), jnp.float32),
    )(*partials, w1, b1, w2, b2)


@functools.cache
def _make_seg_sum(nch):
    rpw = nch * _CH
    mesh = plsc.VectorSubcoreMesh(core_axis_name="c", subcore_axis_name="s")

    @functools.partial(
        pl.kernel,
        mesh=mesh,
        out_type=jax.ShapeDtypeStruct((_NC, _SP, _D), jnp.float32),
        scratch_types=(
            [pltpu.VMEM((1, _CH), jnp.int32) for _ in range(_NBUF)]
            + [pltpu.VMEM((_CH, _D), jnp.float32) for _ in range(_NBUF)]
            + [pltpu.VMEM_SHARED((_SP, _D), jnp.float32)]
            + [pltpu.SemaphoreType.DMA for _ in range(2 * _NBUF)]
        ),
    )
    def seg_sum(pt_hbm, idx3_hbm, zrows_hbm, out_hbm, *scr):
        ibufs = scr[:_NBUF]
        rbufs = scr[_NBUF : 2 * _NBUF]
        seg_sh = scr[2 * _NBUF]
        lsems = scr[2 * _NBUF + 1 : 3 * _NBUF + 1]
        ssems = scr[3 * _NBUF + 1 : 4 * _NBUF + 1]
        c = lax.axis_index("c")
        s = lax.axis_index("s")
        wid = c * _NS + s
        pltpu.sync_copy(zrows_hbm, seg_sh.at[pl.ds(s * _ZR, _ZR)])
        plsc.subcore_barrier()

        base = wid * rpw

        def rows_src(j):
            return pt_hbm.at[pl.ds(base + j * _CH, _CH)]

        def start_loads(j):
            b = j % _NBUF
            pltpu.async_copy(idx3_hbm.at[wid, j], ibufs[b], lsems[b])
            pltpu.async_copy(rows_src(j), rbufs[b], lsems[b])

        def wait_loads(j):
            b = j % _NBUF
            pltpu.make_async_copy(idx3_hbm.at[wid, j], ibufs[b], lsems[b]).wait()
            pltpu.make_async_copy(rows_src(j), rbufs[b], lsems[b]).wait()

        def drain_scatter(j):
            b = j % _NBUF
            pltpu.make_async_copy(rows_src(j), rbufs[b], ssems[b]).wait()

        for j in range(_LEAD):
            start_loads(j)
        for j in range(nch):
            b = j % _NBUF
            wait_loads(j)
            pltpu.async_copy(
                rbufs[b], seg_sh.at[ibufs[b].at[0]], ssems[b], add=True
            )
            if j >= _LEAD:
                drain_scatter(j - _LEAD)
            if j + _LEAD < nch:
                start_loads(j + _LEAD)
        for j in range(nch - _LEAD, nch):
            drain_scatter(j)
        plsc.subcore_barrier()
        pltpu.sync_copy(
            seg_sh.at[pl.ds(s * _ZR, _ZR)], out_hbm.at[c, pl.ds(s * _ZR, _ZR)]
        )

    return seg_sum


def kernel(x, idx, W1p, b1p, W2p, b2p, W1r, b1r, W2r, b2r):
    idx32 = idx.astype(jnp.int32)
    b1p2, b2p2 = b1p.reshape(1, _H), b2p.reshape(1, _D)
    zrows = jnp.zeros((_ZR, _D), jnp.float32)
    partials = []
    start = 0
    for n in _SLABS:
        nch = n // (_NW * _CH)
        idx3 = lax.slice_in_dim(idx32, start, start + n).reshape(
            _NW, nch, 1, _CH
        )
        pt = _point_net(x, W1p, b1p2, W2p, b2p2, start, n)
        partials.append(_make_seg_sum(nch)(pt, idx3, zrows))
        start += n
    return _reduce_net(
        partials, W1r, b1r.reshape(1, _H), W2r, b2r.reshape(1, _D)
    )


# speedup vs baseline: 1.1424x; 1.0516x over previous
import functools

import jax
import jax.numpy as jnp
from jax import lax
from jax.experimental import pallas as pl
from jax.experimental.pallas import tpu as pltpu
from jax.experimental.pallas import tpu_sc as plsc

_N = 320000
_D = 128
_H = 256
_S = 10000

_NC = 2
_NS = 16
_NW = _NC * _NS
_CH = 80
_BN = 16000

_SLABS = (_N,)

_NBUF = 4
_LEAD = 2
_SP = 10112
_ZR = _SP // _NS


def _pn_body(x_ref, w1_ref, b1_ref, w2_ref, b2_ref, o_ref):
    h = jnp.maximum(
        jnp.dot(x_ref[...], w1_ref[...], preferred_element_type=jnp.float32)
        + b1_ref[...],
        0.0,
    )
    o_ref[...] = (
        jnp.dot(h, w2_ref[...], preferred_element_type=jnp.float32) + b2_ref[...]
    )


def _point_net(x, w1, b1, w2, b2, start, n):
    off = start // _BN
    return pl.pallas_call(
        _pn_body,
        grid=(n // _BN,),
        in_specs=[
            pl.BlockSpec((_BN, _D), lambda i: (i + off, 0)),
            pl.BlockSpec((_D, _H), lambda i: (0, 0)),
            pl.BlockSpec((1, _H), lambda i: (0, 0)),
            pl.BlockSpec((_H, _D), lambda i: (0, 0)),
            pl.BlockSpec((1, _D), lambda i: (0, 0)),
        ],
        out_specs=pl.BlockSpec((_BN, _D), lambda i: (i, 0)),
        out_shape=jax.ShapeDtypeStruct((n, _D), jnp.float32),
    )(x, w1, b1, w2, b2)


_BS = 2000


def _rn_body(*refs):
    p_refs, (w1_ref, b1_ref, w2_ref, b2_ref, o_ref) = refs[:-5], refs[-5:]
    seg = sum(p[0] + p[1] for p in p_refs)
    h = jnp.maximum(
        jnp.dot(seg, w1_ref[...], preferred_element_type=jnp.float32) + b1_ref[...],
        0.0,
    )
    o_ref[...] = (
        jnp.dot(h, w2_ref[...], preferred_element_type=jnp.float32) + b2_ref[...]
    )


def _reduce_net(partials, w1, b1, w2, b2):
    return pl.pallas_call(
        _rn_body,
        grid=(_S // _BS,),
        in_specs=(
            [pl.BlockSpec((2, _BS, _D), lambda i: (0, i, 0)) for _ in partials]
            + [
                pl.BlockSpec((_D, _H), lambda i: (0, 0)),
                pl.BlockSpec((1, _H), lambda i: (0, 0)),
                pl.BlockSpec((_H, _D), lambda i: (0, 0)),
                pl.BlockSpec((1, _D), lambda i: (0, 0)),
            ]
        ),
        out_specs=pl.BlockSpec((_BS, _D), lambda i: (i, 0)),
        out_shape=jax.ShapeDtypeStruct((_S, _D), jnp.float32),
    )(*partials, w1, b1, w2, b2)


@functools.cache
def _make_seg_sum(nch):
    rpw = nch * _CH
    mesh = plsc.VectorSubcoreMesh(core_axis_name="c", subcore_axis_name="s")

    @functools.partial(
        pl.kernel,
        mesh=mesh,
        out_type=jax.ShapeDtypeStruct((_NC, _SP, _D), jnp.float32),
        scratch_types=(
            [pltpu.VMEM((1, _CH), jnp.int32) for _ in range(_NBUF)]
            + [pltpu.VMEM((_CH, _D), jnp.float32) for _ in range(_NBUF)]
            + [pltpu.VMEM_SHARED((_SP, _D), jnp.float32)]
            + [pltpu.SemaphoreType.DMA for _ in range(2 * _NBUF)]
        ),
    )
    def seg_sum(pt_hbm, idx3_hbm, zrows_hbm, out_hbm, *scr):
        ibufs = scr[:_NBUF]
        rbufs = scr[_NBUF : 2 * _NBUF]
        seg_sh = scr[2 * _NBUF]
        lsems = scr[2 * _NBUF + 1 : 3 * _NBUF + 1]
        ssems = scr[3 * _NBUF + 1 : 4 * _NBUF + 1]
        c = lax.axis_index("c")
        s = lax.axis_index("s")
        wid = c * _NS + s
        pltpu.sync_copy(zrows_hbm, seg_sh.at[pl.ds(s * _ZR, _ZR)])
        plsc.subcore_barrier()

        base = wid * rpw

        def rows_src(j):
            return pt_hbm.at[pl.ds(base + j * _CH, _CH)]

        def start_loads(j):
            b = j % _NBUF
            pltpu.async_copy(idx3_hbm.at[wid, j], ibufs[b], lsems[b])
            pltpu.async_copy(rows_src(j), rbufs[b], lsems[b])

        def wait_loads(j):
            b = j % _NBUF
            pltpu.make_async_copy(idx3_hbm.at[wid, j], ibufs[b], lsems[b]).wait()
            pltpu.make_async_copy(rows_src(j), rbufs[b], lsems[b]).wait()

        def drain_scatter(j):
            b = j % _NBUF
            pltpu.make_async_copy(rows_src(j), rbufs[b], ssems[b]).wait()

        for j in range(_LEAD):
            start_loads(j)
        for j in range(nch):
            b = j % _NBUF
            wait_loads(j)
            pltpu.async_copy(
                rbufs[b], seg_sh.at[ibufs[b].at[0]], ssems[b], add=True
            )
            if j >= _LEAD:
                drain_scatter(j - _LEAD)
            if j + _LEAD < nch:
                start_loads(j + _LEAD)
        for j in range(nch - _LEAD, nch):
            drain_scatter(j)
        plsc.subcore_barrier()
        pltpu.sync_copy(
            seg_sh.at[pl.ds(s * _ZR, _ZR)], out_hbm.at[c, pl.ds(s * _ZR, _ZR)]
        )

    return seg_sum


def kernel(x, idx, W1p, b1p, W2p, b2p, W1r, b1r, W2r, b2r):
    idx32 = idx.astype(jnp.int32)
    b1p2, b2p2 = b1p.reshape(1, _H), b2p.reshape(1, _D)
    zrows = jnp.zeros((_ZR, _D), jnp.float32)
    partials = []
    start = 0
    for n in _SLABS:
        nch = n // (_NW * _CH)
        idx3 = lax.slice_in_dim(idx32, start, start + n).reshape(
            _NW, nch, 1, _CH
        )
        pt = _point_net(x, W1p, b1p2, W2p, b2p2, start, n)
        partials.append(_make_seg_sum(nch)(pt, idx3, zrows))
        start += n
    return _reduce_net(
        partials, W1r, b1r.reshape(1, _H), W2r, b2r.reshape(1, _D)
    )
